# baseline (speedup 1.0000x reference)
import jax
import jax.numpy as jnp
from jax.experimental import pallas as pl


def _body(x_ref, w_ref, o_ref):
    y = x_ref[...] + w_ref[...][None]
    o_ref[0] = y
    o_ref[1] = y


def kernel(x, embed_weight):
    B, S, D = x.shape
    TS = 512
    out = pl.pallas_call(
        _body,
        grid=(S // TS,),
        in_specs=[
            pl.BlockSpec((B, TS, D), lambda s: (0, s, 0)),
            pl.BlockSpec((TS, D), lambda s: (s, 0)),
        ],
        out_specs=pl.BlockSpec((B, B, TS, D), lambda s: (0, 0, s, 0)),
        out_shape=jax.ShapeDtypeStruct((B, B, S, D), x.dtype),
    )(x, embed_weight)
    return out


# speedup vs baseline: 1.0075x; 1.0075x over previous
import jax
import jax.numpy as jnp
from jax import lax
from jax.experimental import pallas as pl
from jax.experimental.pallas import tpu as pltpu


def _kernel_manual(x, embed_weight):
    B, S, D = x.shape
    TS = 512
    NSTEP = S // TS

    def body(x_ref, w_ref, o_ref, y_ref, sem):
        s = pl.program_id(0)
        slot = lax.rem(s, 2)

        def waits(step):
            sl = lax.rem(step, 2)
            r0 = step * TS
            for i in range(2):
                pltpu.make_async_copy(
                    y_ref.at[sl],
                    o_ref.at[i, :, pl.ds(r0, TS), :],
                    sem.at[sl],
                ).wait()

        @pl.when(s >= 2)
        def _():
            waits(s - 2)

        y_ref[slot] = x_ref[...] + w_ref[...][None]

        for i in range(2):
            pltpu.async_copy(
                y_ref.at[slot],
                o_ref.at[i, :, pl.ds(s * TS, TS), :],
                sem.at[slot],
            )

        @pl.when(s == NSTEP - 1)
        def _():
            waits(s - 1)
            waits(s)

    out = pl.pallas_call(
        body,
        grid=(NSTEP,),
        in_specs=[
            pl.BlockSpec((B, TS, D), lambda s: (0, s, 0)),
            pl.BlockSpec((TS, D), lambda s: (s, 0)),
        ],
        out_specs=pl.BlockSpec(memory_space=pl.ANY),
        out_shape=jax.ShapeDtypeStruct((B, B, S, D), x.dtype),
        scratch_shapes=[
            pltpu.VMEM((2, B, TS, D), jnp.float32),
            pltpu.SemaphoreType.DMA((2,)),
        ],
    )(x, embed_weight)
    return out


def _body(x_ref, w_ref, o_ref):
    y = x_ref[...] + w_ref[...][None]
    o_ref[0] = y
    o_ref[1] = y


def kernel(x, embed_weight):
    return _kernel_manual(x, embed_weight)


def _kernel_auto(x, embed_weight):
    B, S, D = x.shape
    TS = 512
    out = pl.pallas_call(
        _body,
        grid=(S // TS,),
        in_specs=[
            pl.BlockSpec((B, TS, D), lambda s: (0, s, 0)),
            pl.BlockSpec((TS, D), lambda s: (s, 0)),
        ],
        out_specs=pl.BlockSpec((B, B, TS, D), lambda s: (0, 0, s, 0)),
        out_shape=jax.ShapeDtypeStruct((B, B, S, D), x.dtype),
    )(x, embed_weight)
    return out
